# Initial kernel scaffold; baseline (speedup 1.0000x reference)
import jax
import jax.numpy as jnp
from jax.experimental import pallas as pl


def kernel(coords, labels, point_embed_bg, point_embed_fg, no_mask_embed):
    raise NotImplementedError("write your pallas kernel here")


# speedup vs baseline: 1.0514x; 1.0514x over previous
import functools
import math

import jax
import jax.numpy as jnp
from jax.experimental import pallas as pl

_EMBED_DIM = 256
_IMG = 1024


def _fused_body(x_ref, lab_ref, f2_ref, ph_ref, bg_ref, fg_ref, nm_ref,
                sparse_ref, dense_ref):
    b = pl.program_id(0)
    h = pl.program_id(1)

    @pl.when((b == 0) & (h == 0))
    def _sparse():
        x = x_ref[...]
        lab = lab_ref[...]
        f2 = f2_ref[...].reshape(1, 1, _EMBED_DIM)
        ph = ph_ref[...].reshape(1, 1, _EMBED_DIM)
        pe = jnp.sin(x[:, :, None] * f2 + ph)
        emb = jnp.where(lab[:, :, None] >= 1,
                        fg_ref[...].reshape(1, 1, _EMBED_DIM),
                        bg_ref[...].reshape(1, 1, _EMBED_DIM))
        sparse_ref[...] = pe + emb

    dense_ref[...] = jnp.broadcast_to(nm_ref[...], dense_ref.shape)


def kernel(coords, labels, point_embed_bg, point_embed_fg, no_mask_embed):
    B, N, _ = coords.shape
    D = _EMBED_DIM
    HW = _IMG // 4

    x = coords[:, :, 0]
    lab = labels.astype(jnp.int32)

    half = D // 2
    f = (2.0 ** (jnp.arange(half, dtype=jnp.float32) / half)) * jnp.pi
    f2 = jnp.repeat(f, 2).reshape(1, D)
    ph = jnp.tile(jnp.array([0.0, jnp.pi / 2], dtype=jnp.float32),
                  half).reshape(1, D)
    nm = no_mask_embed.reshape(1, D, 1, 1)

    H_BLK = 32
    grid = (B, HW // H_BLK)

    sparse, dense = pl.pallas_call(
        _fused_body,
        grid=grid,
        in_specs=[
            pl.BlockSpec((B, N), lambda b, h: (0, 0)),
            pl.BlockSpec((B, N), lambda b, h: (0, 0)),
            pl.BlockSpec((1, D), lambda b, h: (0, 0)),
            pl.BlockSpec((1, D), lambda b, h: (0, 0)),
            pl.BlockSpec((1, D), lambda b, h: (0, 0)),
            pl.BlockSpec((1, D), lambda b, h: (0, 0)),
            pl.BlockSpec((1, D, 1, 1), lambda b, h: (0, 0, 0, 0)),
        ],
        out_specs=[
            pl.BlockSpec((B, N, D), lambda b, h: (0, 0, 0)),
            pl.BlockSpec((1, D, H_BLK, HW), lambda b, h: (b, 0, h, 0)),
        ],
        out_shape=[
            jax.ShapeDtypeStruct((B, N, D), jnp.float32),
            jax.ShapeDtypeStruct((B, D, HW, HW), jnp.float32),
        ],
    )(x, lab, f2, ph, point_embed_bg, point_embed_fg, nm)
    return (sparse, dense)
